# Initial kernel scaffold; baseline (speedup 1.0000x reference)
import jax
import jax.numpy as jnp
from jax.experimental import pallas as pl


def kernel(solute_x, solute_edge_index, solute_edge_attr, solute_batch, solute_length_matrix, solvent_x, solvent_batch, solvent_length_matrix, params):
    raise NotImplementedError("write your pallas kernel here")


# speedup vs baseline: 3.3597x; 3.3597x over previous
import functools

import jax
import jax.numpy as jnp
from jax import lax
from jax.experimental import pallas as pl
from jax.experimental.pallas import tpu as pltpu
from jax.experimental.pallas import tpu_sc as plsc

EMB = 128
NUM_LAYER = 3
N = 10000
NPAD = 10240
E = 320000
NV = 2000
G = 256
ATOM_F = 40

NC, NS = 2, 16
NW = NC * NS
EPW = E // NW
CH = 40
NCH = EPW // CH
IBLK = 25
NIB = NCH // IBLK
CHC = 80
NCHC = EPW // CHC
CBLK = 25
NCB = NCHC // CBLK
RPS = NPAD // NS


def _sc_aggregate_body(h_hbm, src_hbm, dst_hbm, zero_hbm, out_hbm,
                       src_v, dst_v, rows0, rows1, acc_sh, sem0, sem1):
    c = lax.axis_index("c")
    s = lax.axis_index("s")
    w = s * NC + c
    pltpu.sync_copy(zero_hbm.at[pl.ds(s * RPS, RPS)], acc_sh.at[pl.ds(s * RPS, RPS)])
    plsc.subcore_barrier()

    rows = (rows0, rows1)
    sems = (sem0, sem1)

    def outer(ib, carry):
        pltpu.sync_copy(src_hbm.at[w, ib], src_v)
        pltpu.sync_copy(dst_hbm.at[w, ib], dst_v)
        pltpu.async_copy(h_hbm.at[src_v.at[0]], rows0, sem0)

        def pair(jj, carry2):
            for b in range(2):
                j = jj * 2 + b
                pltpu.async_copy(h_hbm.at[src_v.at[j + 1]], rows[1 - b], sems[1 - b])
                pltpu.make_async_copy(h_hbm.at[src_v.at[j]], rows[b], sems[b]).wait()
                pltpu.sync_copy(rows[b], acc_sh.at[dst_v.at[j]], add=True)
            return carry2

        lax.fori_loop(0, IBLK // 2, pair, 0)
        j = IBLK - 1
        pltpu.make_async_copy(h_hbm.at[src_v.at[j]], rows[j % 2], sems[j % 2]).wait()
        pltpu.sync_copy(rows[j % 2], acc_sh.at[dst_v.at[j]], add=True)
        return carry

    lax.fori_loop(0, NIB, outer, 0)

    plsc.subcore_barrier()
    pltpu.sync_copy(acc_sh.at[pl.ds(s * RPS, RPS)], out_hbm.at[c, pl.ds(s * RPS, RPS)])


@functools.cache
def _sc_kernels():
    mesh = plsc.VectorSubcoreMesh(
        core_axis_name="c", subcore_axis_name="s", num_cores=NC, num_subcores=NS)
    aggregate = pl.kernel(
        _sc_aggregate_body,
        out_type=jax.ShapeDtypeStruct((NC, NPAD, EMB), jnp.float32),
        mesh=mesh,
        scratch_types=[
            pltpu.VMEM((IBLK, CH), jnp.int32),
            pltpu.VMEM((IBLK, CH), jnp.int32),
            pltpu.VMEM((CH, EMB), jnp.float32),
            pltpu.VMEM((CH, EMB), jnp.float32),
            pltpu.VMEM_SHARED((NPAD, EMB), jnp.float32),
            pltpu.SemaphoreType.DMA,
            pltpu.SemaphoreType.DMA,
        ],
    )
    return (aggregate,)


def _gnn_aggregate(table, src4, dst4, zeros_emb):
    return _sc_kernels()[0](table, src4, dst4, zeros_emb)


_RB = 2048


def _dot(a, b, ca=1, cb=0, prec=None):
    return lax.dot_general(a, b, (((ca,), (cb,)), ((), ())),
                           precision=prec,
                           preferred_element_type=jnp.float32)


_HI = lax.Precision.HIGHEST


def _embed_body(x_ref, w_ref, b_ref, o_ref):
    o_ref[...] = jnp.maximum(_dot(x_ref[...], w_ref[...]) + b_ref[...], 0.0)


def _embed(x_pad, w, b):
    return pl.pallas_call(
        _embed_body,
        grid=(NPAD // _RB,),
        in_specs=[
            pl.BlockSpec((_RB, ATOM_F), lambda i: (i, 0)),
            pl.BlockSpec((ATOM_F, EMB), lambda i: (0, 0)),
            pl.BlockSpec((1, EMB), lambda i: (0, 0)),
        ],
        out_specs=pl.BlockSpec((_RB, EMB), lambda i: (i, 0)),
        out_shape=jax.ShapeDtypeStruct((NPAD, EMB), jnp.float32),
    )(x_pad, w, b)


def _solvent_body(x_ref, w1_ref, b1_ref, w2_ref, b2_ref, o_ref):
    h = jnp.maximum(_dot(x_ref[...], w1_ref[...]) + b1_ref[...], 0.0)
    o_ref[...] = _dot(h, w2_ref[...]) + b2_ref[...]


def _solvent_mlp(x, w1, b1, w2, b2):
    return pl.pallas_call(
        _solvent_body,
        out_shape=jax.ShapeDtypeStruct((NV, EMB), jnp.float32),
    )(x, w1, b1, w2, b2)


def _layer_body(mlp_prec, p_ref, h_ref, c0_ref, c1_ref, combo_ref, sc_ref,
                w1_ref, b1_ref, w2_ref, b2_ref, o_ref, ssum_ref, ssq_ref):
    i = pl.program_id(0)
    counts = c0_ref[...] + c1_ref[...]
    p = p_ref[...]
    aggr = (p[0] + p[1] + h_ref[...]
            + _dot(counts, combo_ref[...], prec=_HI) + sc_ref[...])
    hid = jnp.maximum(_dot(aggr, w1_ref[...], prec=mlp_prec) + b1_ref[...], 0.0)
    h2 = _dot(hid, w2_ref[...], prec=mlp_prec) + b2_ref[...]
    o_ref[...] = h2
    row = i * _RB + lax.broadcasted_iota(jnp.int32, (_RB, 1), 0)
    h2m = jnp.where(row < N, h2, 0.0)

    @pl.when(i == 0)
    def _():
        ssum_ref[...] = jnp.zeros_like(ssum_ref)
        ssq_ref[...] = jnp.zeros_like(ssq_ref)

    ssum_ref[...] += jnp.broadcast_to(jnp.sum(h2m, axis=0, keepdims=True), (8, EMB))
    ssq_ref[...] += jnp.broadcast_to(jnp.sum(h2m * h2m, axis=0, keepdims=True), (8, EMB))


def _gin_layer(parts, h, c0, c1, combo, selfc, w1, b1, w2, b2, hi):
    nb = NPAD // _RB
    return pl.pallas_call(
        functools.partial(_layer_body, _HI if hi else None),
        grid=(nb,),
        in_specs=[
            pl.BlockSpec((NC, _RB, EMB), lambda i: (0, i, 0)),
            pl.BlockSpec((_RB, EMB), lambda i: (i, 0)),
            pl.BlockSpec((_RB, EMB), lambda i: (i, 0)),
            pl.BlockSpec((_RB, EMB), lambda i: (i, 0)),
            pl.BlockSpec((EMB, EMB), lambda i: (0, 0)),
            pl.BlockSpec((1, EMB), lambda i: (0, 0)),
            pl.BlockSpec((EMB, 2 * EMB), lambda i: (0, 0)),
            pl.BlockSpec((1, 2 * EMB), lambda i: (0, 0)),
            pl.BlockSpec((2 * EMB, EMB), lambda i: (0, 0)),
            pl.BlockSpec((1, EMB), lambda i: (0, 0)),
        ],
        out_specs=[
            pl.BlockSpec((_RB, EMB), lambda i: (i, 0)),
            pl.BlockSpec((8, EMB), lambda i: (0, 0)),
            pl.BlockSpec((8, EMB), lambda i: (0, 0)),
        ],
        out_shape=[
            jax.ShapeDtypeStruct((NPAD, EMB), jnp.float32),
            jax.ShapeDtypeStruct((8, EMB), jnp.float32),
            jax.ShapeDtypeStruct((8, EMB), jnp.float32),
        ],
    )(parts, h, c0, c1, combo, selfc, w1, b1, w2, b2)


def _norm_body_relu(x_ref, sc_ref, sh_ref, o_ref):
    o_ref[...] = jnp.maximum(x_ref[...] * sc_ref[...] + sh_ref[...], 0.0)


def _norm_body_id(x_ref, sc_ref, sh_ref, o_ref):
    o_ref[...] = x_ref[...] * sc_ref[...] + sh_ref[...]


def _bn_apply(h2, scale, shift, relu):
    return pl.pallas_call(
        _norm_body_relu if relu else _norm_body_id,
        grid=(NPAD // _RB,),
        in_specs=[
            pl.BlockSpec((_RB, EMB), lambda i: (i, 0)),
            pl.BlockSpec((1, EMB), lambda i: (0, 0)),
            pl.BlockSpec((1, EMB), lambda i: (0, 0)),
        ],
        out_specs=pl.BlockSpec((_RB, EMB), lambda i: (i, 0)),
        out_shape=jax.ShapeDtypeStruct((NPAD, EMB), jnp.float32),
    )(h2, scale, shift)


_IB = 1024


def _inter_body(s_ref, ls_ref, lv_ref, v_ref, batch_ref,
                svp_ref, spool_ref):
    i = pl.program_id(0)
    s_blk = s_ref[...]
    v = v_ref[...]
    len_blk = _dot(ls_ref[...], lv_ref[...], ca=0, cb=0)
    m = jnp.tanh(_dot(s_blk, v, ca=1, cb=1)) * len_blk
    sp_blk = _dot(m, v)
    svp_c = _dot(m, s_blk, ca=0, cb=0)
    seg = batch_ref[0, 0, :]
    oh = (lax.broadcasted_iota(jnp.int32, (G, _IB), 0)
          == seg[None, :]).astype(jnp.float32)
    spool_c = jnp.concatenate([_dot(oh, s_blk, prec=_HI),
                               _dot(oh, sp_blk, prec=_HI)], axis=1)

    @pl.when(i == 0)
    def _():
        svp_ref[...] = jnp.zeros_like(svp_ref)
        spool_ref[...] = jnp.zeros_like(spool_ref)

    svp_ref[...] += svp_c
    spool_ref[...] += spool_c


def _interaction(h, ls_pad, lv, vrep, batch3):
    return pl.pallas_call(
        _inter_body,
        grid=(NPAD // _IB,),
        in_specs=[
            pl.BlockSpec((_IB, EMB), lambda i: (i, 0)),
            pl.BlockSpec((G, _IB), lambda i: (0, i)),
            pl.BlockSpec((G, NV), lambda i: (0, 0)),
            pl.BlockSpec((NV, EMB), lambda i: (0, 0)),
            pl.BlockSpec((1, 1, _IB), lambda i: (i, 0, 0)),
        ],
        out_specs=[
            pl.BlockSpec((NV, EMB), lambda i: (0, 0)),
            pl.BlockSpec((G, 2 * EMB), lambda i: (0, 0)),
        ],
        out_shape=[
            jax.ShapeDtypeStruct((NV, EMB), jnp.float32),
            jax.ShapeDtypeStruct((G, 2 * EMB), jnp.float32),
        ],
    )(h, ls_pad, lv, vrep, batch3)


def _final_body(spool_ref, vrep_ref, svp_ref, vb_ref,
                w0_ref, b0_ref, w1_ref, b1_ref, w2_ref, b2_ref, w3_ref, b3_ref,
                o_ref):
    vb = vb_ref[0, 0, :]
    oh = (lax.broadcasted_iota(jnp.int32, (G, NV), 0)
          == vb[None, :]).astype(jnp.float32)
    vpool = jnp.concatenate([_dot(oh, vrep_ref[...], prec=_HI),
                             _dot(oh, svp_ref[...], prec=_HI)], axis=1)
    spool = spool_ref[...]
    x = jnp.concatenate([spool, vpool, spool * vpool], axis=1)
    x = jnp.maximum(_dot(x, w0_ref[...]) + b0_ref[...], 0.0)
    x = jnp.maximum(_dot(x, w1_ref[...]) + b1_ref[...], 0.0)
    x = jnp.maximum(_dot(x, w2_ref[...]) + b2_ref[...], 0.0)
    o_ref[...] = _dot(x, w3_ref[...]) + b3_ref[...]


def _final(spool, vrep, svp, vb3, ro):
    args = [spool, vrep, svp, vb3]
    for p in ro:
        args.append(p['W'])
        args.append(p['b'].reshape(1, -1))
    return pl.pallas_call(
        _final_body,
        out_shape=jax.ShapeDtypeStruct((G, 1), jnp.float32),
    )(*args)


def kernel(solute_x, solute_edge_index, solute_edge_attr, solute_batch,
           solute_length_matrix, solvent_x, solvent_batch,
           solvent_length_matrix, params):
    f32 = jnp.float32
    src4 = solute_edge_index[0].reshape(NW, NIB, IBLK, CH)
    dst4 = solute_edge_index[1].reshape(NW, NIB, IBLK, CH)
    cid4 = (solute_edge_attr[:, 0] * 3 + solute_edge_attr[:, 1]).astype(
        jnp.int32).reshape(NW, NIB, IBLK, CH)
    zeros_emb = jnp.zeros((NPAD, EMB), f32)
    eye_tab = jnp.eye(16, EMB, dtype=f32)

    x_pad = jnp.concatenate(
        [solute_x, jnp.zeros((NPAD - N, ATOM_F), f32)], axis=0)
    batch_pad = jnp.concatenate(
        [solute_batch, jnp.full((NPAD - N,), G, jnp.int32)])
    batch3 = batch_pad.reshape(NPAD // _IB, 1, _IB)
    ls_pad = jnp.concatenate(
        [solute_length_matrix, jnp.zeros((G, NPAD - N), f32)], axis=1)
    vb3 = solvent_batch.reshape(1, 1, NV)

    cparts = _gnn_aggregate(eye_tab, cid4, dst4, zeros_emb)
    c0, c1 = cparts[0], cparts[1]

    h = _embed(x_pad, params['emb']['W'], params['emb']['b'].reshape(1, EMB))
    vrep = _solvent_mlp(solvent_x,
                        params['solv1']['W'], params['solv1']['b'].reshape(1, EMB),
                        params['solv2']['W'], params['solv2']['b'].reshape(1, EMB))

    for l, p in enumerate(params['layers']):
        combo = (p['ee1'][:3][:, None, :] + p['ee2'][None, :3, :]).reshape(9, EMB)
        combo = jnp.concatenate([combo, jnp.zeros((EMB - 9, EMB), f32)], axis=0)
        selfc = (p['ee1'][4] + p['ee2'][0]).reshape(1, EMB)
        parts = _gnn_aggregate(h, src4, dst4, zeros_emb)
        h2, ssum, ssq = _gin_layer(
            parts, h, c0, c1, combo, selfc,
            p['mlp1']['W'], p['mlp1']['b'].reshape(1, -1),
            p['mlp2']['W'], p['mlp2']['b'].reshape(1, -1),
            hi=False)
        mu = ssum[0] / N
        var = ssq[0] / N - mu * mu
        rstd = 1.0 / jnp.sqrt(var + 1e-5)
        scale = (p['bn_g'] * rstd).reshape(1, EMB)
        shift = (p['bn_b'] - mu * p['bn_g'] * rstd).reshape(1, EMB)
        h = _bn_apply(h2, scale, shift, relu=(l < NUM_LAYER - 1))

    svp, spool = _interaction(h, ls_pad, solvent_length_matrix, vrep, batch3)
    return _final(spool, vrep, svp, vb3, params['readout'])


# speedup vs baseline: 7.6651x; 2.2815x over previous
import functools

import jax
import jax.numpy as jnp
from jax import lax
from jax.experimental import pallas as pl
from jax.experimental.pallas import tpu as pltpu
from jax.experimental.pallas import tpu_sc as plsc

EMB = 128
NUM_LAYER = 3
N = 10000
NPAD = 10240
E = 320000
NV = 2000
G = 256
ATOM_F = 40

NC, NS = 2, 16
NW = NC * NS
EPW = E // NW
CH = 40
NCH = EPW // CH
IBLK = 25
NIB = NCH // IBLK
CHC = 80
NCHC = EPW // CHC
CBLK = 25
NCB = NCHC // CBLK
RPS = NPAD // NS


def _sc_aggregate_body(h_hbm, src_hbm, dst_hbm, zero_hbm, out_hbm,
                       src_v, dst_v, rows0, rows1, acc_sh, sem0, sem1):
    c = lax.axis_index("c")
    s = lax.axis_index("s")
    w = s * NC + c
    pltpu.sync_copy(zero_hbm.at[pl.ds(s * RPS, RPS)], acc_sh.at[pl.ds(s * RPS, RPS)])
    plsc.subcore_barrier()

    rows = (rows0, rows1)
    sems = (sem0, sem1)

    def outer(ib, carry):
        pltpu.sync_copy(src_hbm.at[w, ib], src_v)
        pltpu.sync_copy(dst_hbm.at[w, ib], dst_v)
        pltpu.async_copy(h_hbm.at[src_v.at[0]], rows0, sem0)

        def pair(jj, carry2):
            for b in range(2):
                j = jj * 2 + b
                pltpu.async_copy(h_hbm.at[src_v.at[j + 1]], rows[1 - b], sems[1 - b])
                pltpu.make_async_copy(h_hbm.at[src_v.at[j]], rows[b], sems[b]).wait()
                pltpu.sync_copy(rows[b], acc_sh.at[dst_v.at[j]], add=True)
            return carry2

        lax.fori_loop(0, IBLK // 2, pair, 0)
        j = IBLK - 1
        pltpu.make_async_copy(h_hbm.at[src_v.at[j]], rows[j % 2], sems[j % 2]).wait()
        pltpu.sync_copy(rows[j % 2], acc_sh.at[dst_v.at[j]], add=True)
        return carry

    lax.fori_loop(0, NIB, outer, 0)

    plsc.subcore_barrier()
    pltpu.sync_copy(acc_sh.at[pl.ds(s * RPS, RPS)], out_hbm.at[c, pl.ds(s * RPS, RPS)])


@functools.cache
def _sc_kernels():
    mesh = plsc.VectorSubcoreMesh(
        core_axis_name="c", subcore_axis_name="s", num_cores=NC, num_subcores=NS)
    aggregate = pl.kernel(
        _sc_aggregate_body,
        out_type=jax.ShapeDtypeStruct((NC, NPAD, EMB), jnp.float32),
        mesh=mesh,
        scratch_types=[
            pltpu.VMEM((IBLK, CH), jnp.int32),
            pltpu.VMEM((IBLK, CH), jnp.int32),
            pltpu.VMEM((CH, EMB), jnp.float32),
            pltpu.VMEM((CH, EMB), jnp.float32),
            pltpu.VMEM_SHARED((NPAD, EMB), jnp.float32),
            pltpu.SemaphoreType.DMA,
            pltpu.SemaphoreType.DMA,
        ],
    )
    return (aggregate,)


def _gnn_aggregate(table, src4, dst4, zeros_emb):
    return _sc_kernels()[0](table, src4, dst4, zeros_emb)


_RB = 2048


def _dot(a, b, ca=1, cb=0, prec=None):
    return lax.dot_general(a, b, (((ca,), (cb,)), ((), ())),
                           precision=prec,
                           preferred_element_type=jnp.float32)


_HI = lax.Precision.HIGHEST


def _embed_body(x_ref, w_ref, b_ref, o_ref):
    o_ref[...] = jnp.maximum(_dot(x_ref[...], w_ref[...]) + b_ref[...], 0.0)


def _embed(x_pad, w, b):
    return pl.pallas_call(
        _embed_body,
        grid=(NPAD // _RB,),
        in_specs=[
            pl.BlockSpec((_RB, ATOM_F), lambda i: (i, 0)),
            pl.BlockSpec((ATOM_F, EMB), lambda i: (0, 0)),
            pl.BlockSpec((1, EMB), lambda i: (0, 0)),
        ],
        out_specs=pl.BlockSpec((_RB, EMB), lambda i: (i, 0)),
        out_shape=jax.ShapeDtypeStruct((NPAD, EMB), jnp.float32),
    )(x_pad, w, b)


def _solvent_body(x_ref, w1_ref, b1_ref, w2_ref, b2_ref, o_ref):
    h = jnp.maximum(_dot(x_ref[...], w1_ref[...]) + b1_ref[...], 0.0)
    o_ref[...] = _dot(h, w2_ref[...]) + b2_ref[...]


def _solvent_mlp(x, w1, b1, w2, b2):
    return pl.pallas_call(
        _solvent_body,
        out_shape=jax.ShapeDtypeStruct((NV, EMB), jnp.float32),
    )(x, w1, b1, w2, b2)


def _layer_body(mlp_prec, p_ref, h_ref, c0_ref, c1_ref, combo_ref, sc_ref,
                w1_ref, b1_ref, w2_ref, b2_ref, o_ref, ssum_ref, ssq_ref):
    i = pl.program_id(0)
    counts = c0_ref[...] + c1_ref[...]
    p = p_ref[...]
    aggr = (p[0] + p[1] + h_ref[...]
            + _dot(counts, combo_ref[...], prec=_HI) + sc_ref[...])
    hid = jnp.maximum(_dot(aggr, w1_ref[...], prec=mlp_prec) + b1_ref[...], 0.0)
    h2 = _dot(hid, w2_ref[...], prec=mlp_prec) + b2_ref[...]
    o_ref[...] = h2
    row = i * _RB + lax.broadcasted_iota(jnp.int32, (_RB, 1), 0)
    h2m = jnp.where(row < N, h2, 0.0)

    @pl.when(i == 0)
    def _():
        ssum_ref[...] = jnp.zeros_like(ssum_ref)
        ssq_ref[...] = jnp.zeros_like(ssq_ref)

    ssum_ref[...] += jnp.broadcast_to(jnp.sum(h2m, axis=0, keepdims=True), (8, EMB))
    ssq_ref[...] += jnp.broadcast_to(jnp.sum(h2m * h2m, axis=0, keepdims=True), (8, EMB))


def _gin_layer(parts, h, c0, c1, combo, selfc, w1, b1, w2, b2, hi):
    nb = NPAD // _RB
    return pl.pallas_call(
        functools.partial(_layer_body, _HI if hi else None),
        grid=(nb,),
        in_specs=[
            pl.BlockSpec((NC, _RB, EMB), lambda i: (0, i, 0)),
            pl.BlockSpec((_RB, EMB), lambda i: (i, 0)),
            pl.BlockSpec((_RB, EMB), lambda i: (i, 0)),
            pl.BlockSpec((_RB, EMB), lambda i: (i, 0)),
            pl.BlockSpec((EMB, EMB), lambda i: (0, 0)),
            pl.BlockSpec((1, EMB), lambda i: (0, 0)),
            pl.BlockSpec((EMB, 2 * EMB), lambda i: (0, 0)),
            pl.BlockSpec((1, 2 * EMB), lambda i: (0, 0)),
            pl.BlockSpec((2 * EMB, EMB), lambda i: (0, 0)),
            pl.BlockSpec((1, EMB), lambda i: (0, 0)),
        ],
        out_specs=[
            pl.BlockSpec((_RB, EMB), lambda i: (i, 0)),
            pl.BlockSpec((8, EMB), lambda i: (0, 0)),
            pl.BlockSpec((8, EMB), lambda i: (0, 0)),
        ],
        out_shape=[
            jax.ShapeDtypeStruct((NPAD, EMB), jnp.float32),
            jax.ShapeDtypeStruct((8, EMB), jnp.float32),
            jax.ShapeDtypeStruct((8, EMB), jnp.float32),
        ],
    )(parts, h, c0, c1, combo, selfc, w1, b1, w2, b2)


def _norm_body_relu(x_ref, sc_ref, sh_ref, o_ref):
    o_ref[...] = jnp.maximum(x_ref[...] * sc_ref[...] + sh_ref[...], 0.0)


def _norm_body_id(x_ref, sc_ref, sh_ref, o_ref):
    o_ref[...] = x_ref[...] * sc_ref[...] + sh_ref[...]


def _bn_apply(h2, scale, shift, relu):
    return pl.pallas_call(
        _norm_body_relu if relu else _norm_body_id,
        grid=(NPAD // _RB,),
        in_specs=[
            pl.BlockSpec((_RB, EMB), lambda i: (i, 0)),
            pl.BlockSpec((1, EMB), lambda i: (0, 0)),
            pl.BlockSpec((1, EMB), lambda i: (0, 0)),
        ],
        out_specs=pl.BlockSpec((_RB, EMB), lambda i: (i, 0)),
        out_shape=jax.ShapeDtypeStruct((NPAD, EMB), jnp.float32),
    )(h2, scale, shift)


_IB = 1024


def _inter_body(s_ref, ls_ref, lv_ref, v_ref, batch_ref,
                svp_ref, spool_ref):
    i = pl.program_id(0)
    s_blk = s_ref[...]
    v = v_ref[...]
    len_blk = _dot(ls_ref[...], lv_ref[...], ca=0, cb=0)
    m = jnp.tanh(_dot(s_blk, v, ca=1, cb=1)) * len_blk
    sp_blk = _dot(m, v)
    svp_c = _dot(m, s_blk, ca=0, cb=0)
    seg = batch_ref[0, 0, :]
    oh = (lax.broadcasted_iota(jnp.int32, (G, _IB), 0)
          == seg[None, :]).astype(jnp.float32)
    spool_c = jnp.concatenate([_dot(oh, s_blk, prec=_HI),
                               _dot(oh, sp_blk, prec=_HI)], axis=1)

    @pl.when(i == 0)
    def _():
        svp_ref[...] = jnp.zeros_like(svp_ref)
        spool_ref[...] = jnp.zeros_like(spool_ref)

    svp_ref[...] += svp_c
    spool_ref[...] += spool_c


def _interaction(h, ls_pad, lv, vrep, batch3):
    return pl.pallas_call(
        _inter_body,
        grid=(NPAD // _IB,),
        in_specs=[
            pl.BlockSpec((_IB, EMB), lambda i: (i, 0)),
            pl.BlockSpec((G, _IB), lambda i: (0, i)),
            pl.BlockSpec((G, NV), lambda i: (0, 0)),
            pl.BlockSpec((NV, EMB), lambda i: (0, 0)),
            pl.BlockSpec((1, 1, _IB), lambda i: (i, 0, 0)),
        ],
        out_specs=[
            pl.BlockSpec((NV, EMB), lambda i: (0, 0)),
            pl.BlockSpec((G, 2 * EMB), lambda i: (0, 0)),
        ],
        out_shape=[
            jax.ShapeDtypeStruct((NV, EMB), jnp.float32),
            jax.ShapeDtypeStruct((G, 2 * EMB), jnp.float32),
        ],
    )(h, ls_pad, lv, vrep, batch3)


def _final_body(spool_ref, vrep_ref, svp_ref, vb_ref,
                w0_ref, b0_ref, w1_ref, b1_ref, w2_ref, b2_ref, w3_ref, b3_ref,
                o_ref):
    vb = vb_ref[0, 0, :]
    oh = (lax.broadcasted_iota(jnp.int32, (G, NV), 0)
          == vb[None, :]).astype(jnp.float32)
    vpool = jnp.concatenate([_dot(oh, vrep_ref[...], prec=_HI),
                             _dot(oh, svp_ref[...], prec=_HI)], axis=1)
    spool = spool_ref[...]
    x = jnp.concatenate([spool, vpool, spool * vpool], axis=1)
    x = jnp.maximum(_dot(x, w0_ref[...]) + b0_ref[...], 0.0)
    x = jnp.maximum(_dot(x, w1_ref[...]) + b1_ref[...], 0.0)
    x = jnp.maximum(_dot(x, w2_ref[...]) + b2_ref[...], 0.0)
    o_ref[...] = _dot(x, w3_ref[...]) + b3_ref[...]


def _final(spool, vrep, svp, vb3, ro):
    args = [spool, vrep, svp, vb3]
    for p in ro:
        args.append(p['W'])
        args.append(p['b'].reshape(1, -1))
    return pl.pallas_call(
        _final_body,
        out_shape=jax.ShapeDtypeStruct((G, 1), jnp.float32),
    )(*args)


def kernel(solute_x, solute_edge_index, solute_edge_attr, solute_batch,
           solute_length_matrix, solvent_x, solvent_batch,
           solvent_length_matrix, params):
    f32 = jnp.float32
    src4 = solute_edge_index[0].reshape(NW, NIB, IBLK, CH)
    dst4 = solute_edge_index[1].reshape(NW, NIB, IBLK, CH)
    cid = (solute_edge_attr[:, 0] * 3 + solute_edge_attr[:, 1]).astype(jnp.int32)
    cid4 = (cid + 16 * (jnp.arange(E, dtype=jnp.int32) % 128)).reshape(
        NW, NIB, IBLK, CH)
    zeros_emb = jnp.zeros((NPAD, EMB), f32)
    eye_tab = jnp.tile(jnp.eye(16, EMB, dtype=f32), (128, 1))

    x_pad = jnp.concatenate(
        [solute_x, jnp.zeros((NPAD - N, ATOM_F), f32)], axis=0)
    batch_pad = jnp.concatenate(
        [solute_batch, jnp.full((NPAD - N,), G, jnp.int32)])
    batch3 = batch_pad.reshape(NPAD // _IB, 1, _IB)
    ls_pad = jnp.concatenate(
        [solute_length_matrix, jnp.zeros((G, NPAD - N), f32)], axis=1)
    vb3 = solvent_batch.reshape(1, 1, NV)

    cparts = _gnn_aggregate(eye_tab, cid4, dst4, zeros_emb)
    c0, c1 = cparts[0], cparts[1]

    h = _embed(x_pad, params['emb']['W'], params['emb']['b'].reshape(1, EMB))
    vrep = _solvent_mlp(solvent_x,
                        params['solv1']['W'], params['solv1']['b'].reshape(1, EMB),
                        params['solv2']['W'], params['solv2']['b'].reshape(1, EMB))

    for l, p in enumerate(params['layers']):
        combo = (p['ee1'][:3][:, None, :] + p['ee2'][None, :3, :]).reshape(9, EMB)
        combo = jnp.concatenate([combo, jnp.zeros((EMB - 9, EMB), f32)], axis=0)
        selfc = (p['ee1'][4] + p['ee2'][0]).reshape(1, EMB)
        parts = _gnn_aggregate(h, src4, dst4, zeros_emb)
        h2, ssum, ssq = _gin_layer(
            parts, h, c0, c1, combo, selfc,
            p['mlp1']['W'], p['mlp1']['b'].reshape(1, -1),
            p['mlp2']['W'], p['mlp2']['b'].reshape(1, -1),
            hi=False)
        mu = ssum[0] / N
        var = ssq[0] / N - mu * mu
        rstd = 1.0 / jnp.sqrt(var + 1e-5)
        scale = (p['bn_g'] * rstd).reshape(1, EMB)
        shift = (p['bn_b'] - mu * p['bn_g'] * rstd).reshape(1, EMB)
        h = _bn_apply(h2, scale, shift, relu=(l < NUM_LAYER - 1))

    svp, spool = _interaction(h, ls_pad, solvent_length_matrix, vrep, batch3)
    return _final(spool, vrep, svp, vb3, params['readout'])


# speedup vs baseline: 7.6678x; 1.0004x over previous
import functools

import jax
import jax.numpy as jnp
from jax import lax
from jax.experimental import pallas as pl
from jax.experimental.pallas import tpu as pltpu
from jax.experimental.pallas import tpu_sc as plsc

EMB = 128
NUM_LAYER = 3
N = 10000
NPAD = 10240
E = 320000
NV = 2000
G = 256
ATOM_F = 40

NC, NS = 2, 16
NW = NC * NS
EPW = E // NW
CH = 40
NCH = EPW // CH
IBLK = 25
NIB = NCH // IBLK
CHC = 80
NCHC = EPW // CHC
CBLK = 25
NCB = NCHC // CBLK
RPS = NPAD // NS


def _sc_aggregate_body(h_hbm, src_hbm, dst_hbm, zero_hbm, out_hbm,
                       src_v, dst_v, rows0, rows1, acc_sh, sem0, sem1):
    c = lax.axis_index("c")
    s = lax.axis_index("s")
    w = s * NC + c
    pltpu.sync_copy(zero_hbm.at[pl.ds(s * RPS, RPS)], acc_sh.at[pl.ds(s * RPS, RPS)])
    plsc.subcore_barrier()

    rows = (rows0, rows1)
    sems = (sem0, sem1)

    def outer(ib, carry):
        pltpu.sync_copy(src_hbm.at[w, ib], src_v)
        pltpu.sync_copy(dst_hbm.at[w, ib], dst_v)
        pltpu.async_copy(h_hbm.at[src_v.at[0]], rows0, sem0)

        def pair(jj, carry2):
            for b in range(2):
                j = jj * 2 + b
                pltpu.async_copy(h_hbm.at[src_v.at[j + 1]], rows[1 - b], sems[1 - b])
                pltpu.make_async_copy(h_hbm.at[src_v.at[j]], rows[b], sems[b]).wait()
                pltpu.sync_copy(rows[b], acc_sh.at[dst_v.at[j]], add=True)
            return carry2

        lax.fori_loop(0, IBLK // 2, pair, 0)
        j = IBLK - 1
        pltpu.make_async_copy(h_hbm.at[src_v.at[j]], rows[j % 2], sems[j % 2]).wait()
        pltpu.sync_copy(rows[j % 2], acc_sh.at[dst_v.at[j]], add=True)
        return carry

    lax.fori_loop(0, NIB, outer, 0)

    plsc.subcore_barrier()
    pltpu.sync_copy(acc_sh.at[pl.ds(s * RPS, RPS)], out_hbm.at[c, pl.ds(s * RPS, RPS)])


@functools.cache
def _sc_kernels():
    mesh = plsc.VectorSubcoreMesh(
        core_axis_name="c", subcore_axis_name="s", num_cores=NC, num_subcores=NS)
    aggregate = pl.kernel(
        _sc_aggregate_body,
        out_type=jax.ShapeDtypeStruct((NC, NPAD, EMB), jnp.float32),
        mesh=mesh,
        scratch_types=[
            pltpu.VMEM((IBLK, CH), jnp.int32),
            pltpu.VMEM((IBLK, CH), jnp.int32),
            pltpu.VMEM((CH, EMB), jnp.float32),
            pltpu.VMEM((CH, EMB), jnp.float32),
            pltpu.VMEM_SHARED((NPAD, EMB), jnp.float32),
            pltpu.SemaphoreType.DMA,
            pltpu.SemaphoreType.DMA,
        ],
    )
    return (aggregate,)


def _gnn_aggregate(table, src4, dst4, zeros_emb):
    return _sc_kernels()[0](table, src4, dst4, zeros_emb)


_RB = 2048


def _dot(a, b, ca=1, cb=0, prec=None):
    return lax.dot_general(a, b, (((ca,), (cb,)), ((), ())),
                           precision=prec,
                           preferred_element_type=jnp.float32)


_HI = lax.Precision.HIGHEST


def _embed_body(x_ref, w_ref, b_ref, o_ref):
    o_ref[...] = jnp.maximum(_dot(x_ref[...], w_ref[...]) + b_ref[...], 0.0)


def _embed(x_pad, w, b):
    return pl.pallas_call(
        _embed_body,
        grid=(NPAD // _RB,),
        in_specs=[
            pl.BlockSpec((_RB, ATOM_F), lambda i: (i, 0)),
            pl.BlockSpec((ATOM_F, EMB), lambda i: (0, 0)),
            pl.BlockSpec((1, EMB), lambda i: (0, 0)),
        ],
        out_specs=pl.BlockSpec((_RB, EMB), lambda i: (i, 0)),
        out_shape=jax.ShapeDtypeStruct((NPAD, EMB), jnp.float32),
    )(x_pad, w, b)


def _solvent_body(x_ref, w1_ref, b1_ref, w2_ref, b2_ref, o_ref):
    h = jnp.maximum(_dot(x_ref[...], w1_ref[...]) + b1_ref[...], 0.0)
    o_ref[...] = _dot(h, w2_ref[...]) + b2_ref[...]


def _solvent_mlp(x, w1, b1, w2, b2):
    return pl.pallas_call(
        _solvent_body,
        out_shape=jax.ShapeDtypeStruct((NV, EMB), jnp.float32),
    )(x, w1, b1, w2, b2)


def _layer_body(p_ref, h_ref, c0_ref, c1_ref, combo_ref, sc_ref,
                w1_ref, b1_ref, w2_ref, b2_ref, o_ref, ssum_ref, ssq_ref):
    i = pl.program_id(0)
    counts = c0_ref[...] + c1_ref[...]
    p = p_ref[...]
    aggr = (p[0] + p[1] + h_ref[...]
            + _dot(counts, combo_ref[...], prec=_HI) + sc_ref[...])
    hid = jnp.maximum(_dot(aggr, w1_ref[...]) + b1_ref[...], 0.0)
    h2 = _dot(hid, w2_ref[...]) + b2_ref[...]
    o_ref[...] = h2
    row = i * _RB + lax.broadcasted_iota(jnp.int32, (_RB, 1), 0)
    h2m = jnp.where(row < N, h2, 0.0)

    @pl.when(i == 0)
    def _():
        ssum_ref[...] = jnp.zeros_like(ssum_ref)
        ssq_ref[...] = jnp.zeros_like(ssq_ref)

    ssum_ref[...] += jnp.broadcast_to(jnp.sum(h2m, axis=0, keepdims=True), (8, EMB))
    ssq_ref[...] += jnp.broadcast_to(jnp.sum(h2m * h2m, axis=0, keepdims=True), (8, EMB))


def _gin_layer(parts, h, c0, c1, combo, selfc, w1, b1, w2, b2):
    nb = NPAD // _RB
    return pl.pallas_call(
        _layer_body,
        grid=(nb,),
        in_specs=[
            pl.BlockSpec((NC, _RB, EMB), lambda i: (0, i, 0)),
            pl.BlockSpec((_RB, EMB), lambda i: (i, 0)),
            pl.BlockSpec((_RB, EMB), lambda i: (i, 0)),
            pl.BlockSpec((_RB, EMB), lambda i: (i, 0)),
            pl.BlockSpec((EMB, EMB), lambda i: (0, 0)),
            pl.BlockSpec((1, EMB), lambda i: (0, 0)),
            pl.BlockSpec((EMB, 2 * EMB), lambda i: (0, 0)),
            pl.BlockSpec((1, 2 * EMB), lambda i: (0, 0)),
            pl.BlockSpec((2 * EMB, EMB), lambda i: (0, 0)),
            pl.BlockSpec((1, EMB), lambda i: (0, 0)),
        ],
        out_specs=[
            pl.BlockSpec((_RB, EMB), lambda i: (i, 0)),
            pl.BlockSpec((8, EMB), lambda i: (0, 0)),
            pl.BlockSpec((8, EMB), lambda i: (0, 0)),
        ],
        out_shape=[
            jax.ShapeDtypeStruct((NPAD, EMB), jnp.float32),
            jax.ShapeDtypeStruct((8, EMB), jnp.float32),
            jax.ShapeDtypeStruct((8, EMB), jnp.float32),
        ],
    )(parts, h, c0, c1, combo, selfc, w1, b1, w2, b2)


def _norm_body_relu(x_ref, mu_ref, s_ref, g_ref, b_ref, o_ref):
    r = (x_ref[...] - mu_ref[...]) / s_ref[...] * g_ref[...] + b_ref[...]
    o_ref[...] = jnp.maximum(r, 0.0)


def _norm_body_id(x_ref, mu_ref, s_ref, g_ref, b_ref, o_ref):
    o_ref[...] = (x_ref[...] - mu_ref[...]) / s_ref[...] * g_ref[...] + b_ref[...]


def _bn_apply(h2, mu, s, g, b, relu):
    return pl.pallas_call(
        _norm_body_relu if relu else _norm_body_id,
        grid=(NPAD // _RB,),
        in_specs=[
            pl.BlockSpec((_RB, EMB), lambda i: (i, 0)),
            pl.BlockSpec((1, EMB), lambda i: (0, 0)),
            pl.BlockSpec((1, EMB), lambda i: (0, 0)),
            pl.BlockSpec((1, EMB), lambda i: (0, 0)),
            pl.BlockSpec((1, EMB), lambda i: (0, 0)),
        ],
        out_specs=pl.BlockSpec((_RB, EMB), lambda i: (i, 0)),
        out_shape=jax.ShapeDtypeStruct((NPAD, EMB), jnp.float32),
    )(h2, mu, s, g, b)


_IB = 1024


def _inter_body(s_ref, ls_ref, lv_ref, v_ref, batch_ref,
                svp_ref, spool_ref):
    i = pl.program_id(0)
    s_blk = s_ref[...]
    v = v_ref[...]
    len_blk = _dot(ls_ref[...], lv_ref[...], ca=0, cb=0)
    m = jnp.tanh(_dot(s_blk, v, ca=1, cb=1)) * len_blk
    sp_blk = _dot(m, v)
    svp_c = _dot(m, s_blk, ca=0, cb=0)
    seg = batch_ref[0, 0, :]
    oh = (lax.broadcasted_iota(jnp.int32, (G, _IB), 0)
          == seg[None, :]).astype(jnp.float32)
    spool_c = jnp.concatenate([_dot(oh, s_blk, prec=_HI),
                               _dot(oh, sp_blk, prec=_HI)], axis=1)

    @pl.when(i == 0)
    def _():
        svp_ref[...] = jnp.zeros_like(svp_ref)
        spool_ref[...] = jnp.zeros_like(spool_ref)

    svp_ref[...] += svp_c
    spool_ref[...] += spool_c


def _interaction(h, ls_pad, lv, vrep, batch3):
    return pl.pallas_call(
        _inter_body,
        grid=(NPAD // _IB,),
        in_specs=[
            pl.BlockSpec((_IB, EMB), lambda i: (i, 0)),
            pl.BlockSpec((G, _IB), lambda i: (0, i)),
            pl.BlockSpec((G, NV), lambda i: (0, 0)),
            pl.BlockSpec((NV, EMB), lambda i: (0, 0)),
            pl.BlockSpec((1, 1, _IB), lambda i: (i, 0, 0)),
        ],
        out_specs=[
            pl.BlockSpec((NV, EMB), lambda i: (0, 0)),
            pl.BlockSpec((G, 2 * EMB), lambda i: (0, 0)),
        ],
        out_shape=[
            jax.ShapeDtypeStruct((NV, EMB), jnp.float32),
            jax.ShapeDtypeStruct((G, 2 * EMB), jnp.float32),
        ],
    )(h, ls_pad, lv, vrep, batch3)


def _final_body(spool_ref, vrep_ref, svp_ref, vb_ref,
                w0_ref, b0_ref, w1_ref, b1_ref, w2_ref, b2_ref, w3_ref, b3_ref,
                o_ref):
    vb = vb_ref[0, 0, :]
    oh = (lax.broadcasted_iota(jnp.int32, (G, NV), 0)
          == vb[None, :]).astype(jnp.float32)
    vpool = jnp.concatenate([_dot(oh, vrep_ref[...], prec=_HI),
                             _dot(oh, svp_ref[...], prec=_HI)], axis=1)
    spool = spool_ref[...]
    x = jnp.concatenate([spool, vpool, spool * vpool], axis=1)
    x = jnp.maximum(_dot(x, w0_ref[...]) + b0_ref[...], 0.0)
    x = jnp.maximum(_dot(x, w1_ref[...]) + b1_ref[...], 0.0)
    x = jnp.maximum(_dot(x, w2_ref[...]) + b2_ref[...], 0.0)
    o_ref[...] = _dot(x, w3_ref[...]) + b3_ref[...]


def _final(spool, vrep, svp, vb3, ro):
    args = [spool, vrep, svp, vb3]
    for p in ro:
        args.append(p['W'])
        args.append(p['b'].reshape(1, -1))
    return pl.pallas_call(
        _final_body,
        out_shape=jax.ShapeDtypeStruct((G, 1), jnp.float32),
    )(*args)


def kernel(solute_x, solute_edge_index, solute_edge_attr, solute_batch,
           solute_length_matrix, solvent_x, solvent_batch,
           solvent_length_matrix, params):
    f32 = jnp.float32
    src4 = solute_edge_index[0].reshape(NW, NIB, IBLK, CH)
    dst4 = solute_edge_index[1].reshape(NW, NIB, IBLK, CH)
    cid = (solute_edge_attr[:, 0] * 3 + solute_edge_attr[:, 1]).astype(jnp.int32)
    cid4 = (cid + 16 * (jnp.arange(E, dtype=jnp.int32) % 128)).reshape(
        NW, NIB, IBLK, CH)
    zeros_emb = jnp.zeros((NPAD, EMB), f32)
    eye_tab = jnp.tile(jnp.eye(16, EMB, dtype=f32), (128, 1))

    x_pad = jnp.concatenate(
        [solute_x, jnp.zeros((NPAD - N, ATOM_F), f32)], axis=0)
    batch_pad = jnp.concatenate(
        [solute_batch, jnp.full((NPAD - N,), G, jnp.int32)])
    batch3 = batch_pad.reshape(NPAD // _IB, 1, _IB)
    ls_pad = jnp.concatenate(
        [solute_length_matrix, jnp.zeros((G, NPAD - N), f32)], axis=1)
    vb3 = solvent_batch.reshape(1, 1, NV)

    cparts = _gnn_aggregate(eye_tab, cid4, dst4, zeros_emb)
    c0, c1 = cparts[0], cparts[1]

    h = _embed(x_pad, params['emb']['W'], params['emb']['b'].reshape(1, EMB))
    vrep = _solvent_mlp(solvent_x,
                        params['solv1']['W'], params['solv1']['b'].reshape(1, EMB),
                        params['solv2']['W'], params['solv2']['b'].reshape(1, EMB))

    for l, p in enumerate(params['layers']):
        combo = (p['ee1'][:3][:, None, :] + p['ee2'][None, :3, :]).reshape(9, EMB)
        combo = jnp.concatenate([combo, jnp.zeros((EMB - 9, EMB), f32)], axis=0)
        selfc = (p['ee1'][4] + p['ee2'][0]).reshape(1, EMB)
        parts = _gnn_aggregate(h, src4, dst4, zeros_emb)
        h2, ssum, ssq = _gin_layer(
            parts, h, c0, c1, combo, selfc,
            p['mlp1']['W'], p['mlp1']['b'].reshape(1, -1),
            p['mlp2']['W'], p['mlp2']['b'].reshape(1, -1))
        mu = ssum[0] / N
        var = ssq[0] / N - mu * mu
        s = jnp.sqrt(var + 1e-5)
        h = _bn_apply(h2, mu.reshape(1, EMB), s.reshape(1, EMB),
                      p['bn_g'].reshape(1, EMB), p['bn_b'].reshape(1, EMB),
                      relu=(l < NUM_LAYER - 1))

    svp, spool = _interaction(h, ls_pad, solvent_length_matrix, vrep, batch3)
    return _final(spool, vrep, svp, vb3, params['readout'])
